# Initial kernel scaffold; baseline (speedup 1.0000x reference)
import jax
import jax.numpy as jnp
from jax.experimental import pallas as pl


def kernel(features, coords, instance_ids, W1, W2, W3, b3):
    raise NotImplementedError("write your pallas kernel here")


# speedup vs baseline: 3.7308x; 3.7308x over previous
import functools

import jax
import jax.numpy as jnp
from jax import lax
from jax.experimental import pallas as pl
from jax.experimental.pallas import tpu as pltpu
from jax.experimental.pallas import tpu_sc as plsc

N, D, K, OUT = 160000, 256, 128, 64
NC, NS = 2, 16
NW = NC * NS
SPW = K // NW
CH = 128
NV = D // 16

BN = 2000
NB = N // BN

_mesh = plsc.VectorSubcoreMesh(core_axis_name="c", subcore_axis_name="s",
                               num_cores=NC, num_subcores=NS)


@functools.partial(
    pl.kernel,
    out_type=jax.ShapeDtypeStruct((K, D), jnp.float32),
    mesh=_mesh,
    compiler_params=pltpu.CompilerParams(use_tc_tiling_on_sc=False),
    scratch_types=[
        pltpu.VMEM((CH, D), jnp.float32),
        pltpu.VMEM((256,), jnp.int32),
        pltpu.VMEM((SPW, D), jnp.float32),
    ],
)
def _sc_segment_sum(feat_hbm, bnd_hbm, fsum_hbm, fbuf, bndv, ostage):
  c = lax.axis_index("c")
  sub = lax.axis_index("s")
  wid = c * NS + sub

  pltpu.sync_copy(bnd_hbm, bndv)

  for s in range(SPW):
    k = SPW * wid + s
    bv = bndv[pl.ds(k, 16)]
    rs = bv[0]
    re = bv[1]
    a8 = (rs >> 3) << 3
    nch = lax.div(re - a8 + (CH - 1), CH)

    def chunk_body(g, accs, rs=rs, re=re, a8=a8):
      cs0 = a8 + g * CH
      cs = jnp.minimum(cs0, N - CH)
      pltpu.sync_copy(feat_hbm.at[pl.ds(cs, CH)], fbuf)
      lo = jnp.maximum(rs, cs0) - cs
      hi = jnp.minimum(re, cs0 + CH) - cs

      def row_body(r, accs):
        return tuple(accs[t] + fbuf[r, pl.ds(16 * t, 16)] for t in range(NV))

      return lax.fori_loop(lo, hi, row_body, accs)

    accs = lax.fori_loop(
        0, nch, chunk_body,
        tuple(jnp.zeros((16,), jnp.float32) for _ in range(NV)))
    for t in range(NV):
      ostage[s, pl.ds(16 * t, 16)] = accs[t]

  pltpu.sync_copy(ostage, fsum_hbm.at[pl.ds(SPW * wid, SPW)])


def _tc_bnd_body(ids_ref, bnd_ref):
  i = pl.program_id(0)

  @pl.when(i == 0)
  def _():
    bnd_ref[...] = jnp.zeros_like(bnd_ref)

  ids = ids_ref[0, 0]
  ltk = (ids[:, None]
         < lax.broadcasted_iota(jnp.int32, (BN, 256), 1)).astype(jnp.int32)
  bnd_ref[...] += jnp.sum(ltk, axis=0, keepdims=True)


def _tc_head_body(ids_ref, c4_ref, fs_ref, w1_ref, w2_ref, w3_ref, b3_ref,
                  emb_ref, cent_ref, out_ref, acc_ref):
  i = pl.program_id(0)

  @pl.when(i == 0)
  def _():
    acc_ref[...] = jnp.zeros_like(acc_ref)

  ids = ids_ref[0, 0]
  oh = (lax.broadcasted_iota(jnp.int32, (K, BN), 0)
        == ids[None, :]).astype(jnp.float32)
  acc_ref[...] += lax.dot_general(
      oh, c4_ref[...], (((1,), (0,)), ((), ())),
      preferred_element_type=jnp.float32)

  @pl.when(i == NB - 1)
  def _():
    aux = acc_ref[...]
    inv = 1.0 / jnp.maximum(aux[:, 3:4], 1.0)
    emb = fs_ref[...] * inv
    emb_ref[...] = emb
    cent_ref[...] = aux[:, 0:3] * inv
    h = jax.nn.relu(jnp.dot(emb, w1_ref[...],
                            preferred_element_type=jnp.float32))
    h = jax.nn.relu(jnp.dot(h, w2_ref[...],
                            preferred_element_type=jnp.float32))
    out_ref[...] = (jnp.dot(h, w3_ref[...],
                            preferred_element_type=jnp.float32) + b3_ref[...])


def kernel(features, coords, instance_ids, W1, W2, W3, b3):
  ids = instance_ids.astype(jnp.int32)
  ids3 = ids.reshape(NB, 1, BN)

  bnd2 = pl.pallas_call(
      _tc_bnd_body,
      grid=(NB,),
      in_specs=[pl.BlockSpec((1, 1, BN), lambda i: (i, 0, 0))],
      out_specs=pl.BlockSpec((1, 256), lambda i: (0, 0)),
      out_shape=jax.ShapeDtypeStruct((1, 256), jnp.int32),
  )(ids3)

  fsum = _sc_segment_sum(features, bnd2[0])

  crd4 = jnp.concatenate(
      [coords.astype(jnp.float32), jnp.ones((N, 1), jnp.float32)], axis=1)
  emb, cent, out = pl.pallas_call(
      _tc_head_body,
      grid=(NB,),
      in_specs=[
          pl.BlockSpec((1, 1, BN), lambda i: (i, 0, 0)),
          pl.BlockSpec((BN, 4), lambda i: (i, 0)),
          pl.BlockSpec((K, D), lambda i: (0, 0)),
          pl.BlockSpec((D, 64), lambda i: (0, 0)),
          pl.BlockSpec((64, 64), lambda i: (0, 0)),
          pl.BlockSpec((64, OUT), lambda i: (0, 0)),
          pl.BlockSpec((OUT,), lambda i: (0,)),
      ],
      out_specs=[
          pl.BlockSpec((K, D), lambda i: (0, 0)),
          pl.BlockSpec((K, 3), lambda i: (0, 0)),
          pl.BlockSpec((K, OUT), lambda i: (0, 0)),
      ],
      out_shape=[jax.ShapeDtypeStruct((K, D), jnp.float32),
                 jax.ShapeDtypeStruct((K, 3), jnp.float32),
                 jax.ShapeDtypeStruct((K, OUT), jnp.float32)],
      scratch_shapes=[pltpu.VMEM((K, 4), jnp.float32)],
  )(ids3, crd4, fsum, W1, W2, W3, b3)
  return emb, cent, out


# speedup vs baseline: 5.0915x; 1.3647x over previous
import functools

import jax
import jax.numpy as jnp
from jax import lax
from jax.experimental import pallas as pl
from jax.experimental.pallas import tpu as pltpu
from jax.experimental.pallas import tpu_sc as plsc

N, D, K, OUT = 160000, 256, 128, 64
NC, NS = 2, 16
NW = NC * NS
SPW = K // NW
CH = 128
NV = D // 16

BN = 2000
NB = N // BN

_mesh = plsc.VectorSubcoreMesh(core_axis_name="c", subcore_axis_name="s",
                               num_cores=NC, num_subcores=NS)


@functools.partial(
    pl.kernel,
    out_type=jax.ShapeDtypeStruct((NW, SPW, D), jnp.float32),
    mesh=_mesh,
    scratch_types=[
        pltpu.VMEM((CH, D), jnp.float32),
        pltpu.VMEM((CH, D), jnp.float32),
        pltpu.VMEM((256,), jnp.int32),
        pltpu.VMEM((SPW, D), jnp.float32),
        pltpu.SemaphoreType.DMA,
        pltpu.SemaphoreType.DMA,
    ],
)
def _sc_segment_sum(feat_hbm, bnd_hbm, fsum_hbm, fbuf0, fbuf1, bndv, ostage,
                    sem0, sem1):
  c = lax.axis_index("c")
  sub = lax.axis_index("s")
  wid = c * NS + sub
  fbufs = (fbuf0, fbuf1)
  sems = (sem0, sem1)

  pltpu.sync_copy(bnd_hbm, bndv)

  for s in range(SPW):
    k = SPW * wid + s
    bv = bndv[pl.ds(k, 16)]
    rs = bv[0]
    re = bv[1]
    a8 = (rs >> 3) << 3
    nch = lax.div(re - a8 + (CH - 1), CH)

    def chunk_start(g, slot, nch=nch, a8=a8):
      @pl.when(g < nch)
      def _():
        cs = pl.multiple_of(jnp.minimum(a8 + g * CH, N - CH), 8)
        pltpu.async_copy(feat_hbm.at[pl.ds(cs, CH)], fbufs[slot], sems[slot])

    def chunk_wait(g, slot, nch=nch, a8=a8):
      @pl.when(g < nch)
      def _():
        cs = pl.multiple_of(jnp.minimum(a8 + g * CH, N - CH), 8)
        pltpu.make_async_copy(feat_hbm.at[pl.ds(cs, CH)],
                              fbufs[slot], sems[slot]).wait()

    def chunk_rows(g, slot, accs, rs=rs, re=re, a8=a8):
      cs0 = a8 + g * CH
      cs = jnp.minimum(cs0, N - CH)
      lo = jnp.maximum(rs, cs0) - cs
      hi = jnp.minimum(re, cs0 + CH) - cs
      fb = fbufs[slot]

      def row_body(r, accs):
        return tuple(accs[t] + fb[r, pl.ds(16 * t, 16)] for t in range(NV))

      return lax.fori_loop(lo, hi, row_body, accs)

    def pair_body(j, accs):
      g0 = 2 * j
      chunk_start(g0 + 1, 1)
      chunk_wait(g0, 0)
      accs = chunk_rows(g0, 0, accs)
      chunk_start(g0 + 2, 0)
      chunk_wait(g0 + 1, 1)
      return chunk_rows(g0 + 1, 1, accs)

    chunk_start(0, 0)
    accs = lax.fori_loop(
        0, lax.div(nch + 1, 2), pair_body,
        tuple(jnp.zeros((16,), jnp.float32) for _ in range(NV)))
    for t in range(NV):
      ostage[s, pl.ds(16 * t, 16)] = accs[t]

  pltpu.sync_copy(ostage, fsum_hbm.at[wid])


def _tc_bnd_body(ids_ref, bnd_ref):
  i = pl.program_id(0)

  @pl.when(i == 0)
  def _():
    bnd_ref[...] = jnp.zeros_like(bnd_ref)

  ids = ids_ref[0, 0]
  ltk = (ids[:, None]
         < lax.broadcasted_iota(jnp.int32, (BN, 256), 1)).astype(jnp.int32)
  bnd_ref[...] += jnp.sum(ltk, axis=0, keepdims=True)


def _tc_head_body(ids_ref, c4_ref, fs_ref, w1_ref, w2_ref, w3_ref, b3_ref,
                  emb_ref, cent_ref, out_ref, acc_ref):
  i = pl.program_id(0)

  @pl.when(i == 0)
  def _():
    acc_ref[...] = jnp.zeros_like(acc_ref)

  ids = ids_ref[0, 0]
  oh = (lax.broadcasted_iota(jnp.int32, (K, BN), 0)
        == ids[None, :]).astype(jnp.float32)
  acc_ref[...] += lax.dot_general(
      oh, c4_ref[...], (((1,), (0,)), ((), ())),
      preferred_element_type=jnp.float32)

  @pl.when(i == NB - 1)
  def _():
    aux = acc_ref[...]
    inv = 1.0 / jnp.maximum(aux[:, 3:4], 1.0)
    emb = fs_ref[...] * inv
    emb_ref[...] = emb
    cent_ref[...] = aux[:, 0:3] * inv
    h = jax.nn.relu(jnp.dot(emb, w1_ref[...],
                            preferred_element_type=jnp.float32))
    h = jax.nn.relu(jnp.dot(h, w2_ref[...],
                            preferred_element_type=jnp.float32))
    out_ref[...] = (jnp.dot(h, w3_ref[...],
                            preferred_element_type=jnp.float32) + b3_ref[...])


def kernel(features, coords, instance_ids, W1, W2, W3, b3):
  ids = instance_ids.astype(jnp.int32)
  ids3 = ids.reshape(NB, 1, BN)

  bnd2 = pl.pallas_call(
      _tc_bnd_body,
      grid=(NB,),
      in_specs=[pl.BlockSpec((1, 1, BN), lambda i: (i, 0, 0))],
      out_specs=pl.BlockSpec((1, 256), lambda i: (0, 0)),
      out_shape=jax.ShapeDtypeStruct((1, 256), jnp.int32),
  )(ids3)

  fsum = _sc_segment_sum(features, bnd2[0]).reshape(K, D)

  crd4 = jnp.concatenate(
      [coords.astype(jnp.float32), jnp.ones((N, 1), jnp.float32)], axis=1)
  emb, cent, out = pl.pallas_call(
      _tc_head_body,
      grid=(NB,),
      in_specs=[
          pl.BlockSpec((1, 1, BN), lambda i: (i, 0, 0)),
          pl.BlockSpec((BN, 4), lambda i: (i, 0)),
          pl.BlockSpec((K, D), lambda i: (0, 0)),
          pl.BlockSpec((D, 64), lambda i: (0, 0)),
          pl.BlockSpec((64, 64), lambda i: (0, 0)),
          pl.BlockSpec((64, OUT), lambda i: (0, 0)),
          pl.BlockSpec((OUT,), lambda i: (0,)),
      ],
      out_specs=[
          pl.BlockSpec((K, D), lambda i: (0, 0)),
          pl.BlockSpec((K, 3), lambda i: (0, 0)),
          pl.BlockSpec((K, OUT), lambda i: (0, 0)),
      ],
      out_shape=[jax.ShapeDtypeStruct((K, D), jnp.float32),
                 jax.ShapeDtypeStruct((K, 3), jnp.float32),
                 jax.ShapeDtypeStruct((K, OUT), jnp.float32)],
      scratch_shapes=[pltpu.VMEM((K, 4), jnp.float32)],
  )(ids3, crd4, fsum, W1, W2, W3, b3)
  return emb, cent, out
